# baseline (speedup 1.0000x reference)
import math

import jax
import jax.numpy as jnp
import numpy as np
from jax import lax
from jax.experimental import pallas as pl
from jax.experimental.pallas import tpu as pltpu

S = 2048
H = 1024
NH = 16
NKV = 8
HD = 64
I = 3584
E = 8
K = 2
THETA = 10000.0
EPS = 1e-06
NEG = float(jnp.finfo(jnp.float32).min)

_BS1 = 256
_BQ = 256
_BS3 = 512
_BI = 256


def _dot_t(a, b):
    return lax.dot_general(a, b, (((1,), (1,)), ((), ())),
                           preferred_element_type=jnp.float32)


def _qkv_body(h_ref, ln1_ref, cq_ref, sq_ref, ck_ref, sk_ref,
              wq_ref, wk_ref, wv_ref, q_ref, k_ref, v_ref):
    x = h_ref[...]
    var = jnp.mean(x * x, axis=1, keepdims=True)
    xn = (x * lax.rsqrt(var + EPS)) * ln1_ref[...]
    q = _dot_t(xn, wq_ref[...])
    k = _dot_t(xn, wk_ref[...])
    v = _dot_t(xn, wv_ref[...])
    hq = NH * 32
    qrot = jnp.concatenate([-q[:, hq:], q[:, :hq]], axis=1)
    q_ref[...] = q * cq_ref[...] + qrot * sq_ref[...]
    hk = NKV * 32
    krot = jnp.concatenate([-k[:, hk:], k[:, :hk]], axis=1)
    k_ref[...] = k * ck_ref[...] + krot * sk_ref[...]
    v_ref[...] = v


def _attn_body(q_ref, k_ref, v_ref, o_ref):
    qb = pl.program_id(1)
    q = q_ref[0]
    k = k_ref[0]
    s = _dot_t(q, k) * (1.0 / math.sqrt(HD))
    row = qb * _BQ + lax.broadcasted_iota(jnp.int32, (_BQ, S), 0)
    col = lax.broadcasted_iota(jnp.int32, (_BQ, S), 1)
    s = jnp.where(col <= row, s, NEG)
    m = jnp.max(s, axis=1, keepdims=True)
    p = jnp.exp(s - m)
    p = p / jnp.sum(p, axis=1, keepdims=True)
    o_ref[0] = jnp.dot(p, v_ref[0], preferred_element_type=jnp.float32)


def _out_router_body(ctx_ref, h_ref, wo_ref, ln2_ref, gw_ref, gb_ref,
                     h1_ref, x2_ref, pw_ref):
    attn_out = _dot_t(ctx_ref[...], wo_ref[...])
    h1 = h_ref[...] + attn_out
    h1_ref[...] = h1
    var = jnp.mean(h1 * h1, axis=1, keepdims=True)
    x2 = (h1 * lax.rsqrt(var + EPS)) * ln2_ref[...]
    x2_ref[...] = x2
    logits = _dot_t(x2, gw_ref[...]) + gb_ref[...]
    mx = jnp.max(logits, axis=1, keepdims=True)
    ex = jnp.exp(logits - mx)
    probs = ex / jnp.sum(ex, axis=1, keepdims=True)
    idx = lax.broadcasted_iota(jnp.int32, probs.shape, 1)
    m1 = jnp.max(probs, axis=1, keepdims=True)
    c1 = jnp.where(probs == m1, idx, E)
    i1 = jnp.min(c1, axis=1, keepdims=True)
    p2 = jnp.where(idx == i1, -1.0, probs)
    m2 = jnp.max(p2, axis=1, keepdims=True)
    c2 = jnp.where(p2 == m2, idx, E)
    i2 = jnp.min(c2, axis=1, keepdims=True)
    pw_ref[...] = jnp.where(idx == i1, m1, jnp.where(idx == i2, m2, 0.0))


def _moe_body(x2_ref, h1_ref, pw_ref, w1_ref, w3_ref, w2_ref, o_ref):
    e = pl.program_id(0)
    i = pl.program_id(1)
    x2 = x2_ref[...]
    a1 = _dot_t(x2, w1_ref[0])
    a3 = _dot_t(x2, w3_ref[0])
    g = (a1 / (1.0 + jnp.exp(-a1))) * a3
    part = _dot_t(g, w2_ref[0])
    sel = lax.broadcasted_iota(jnp.int32, (S, E), 1) == e
    col = jnp.sum(jnp.where(sel, pw_ref[...], 0.0), axis=1, keepdims=True)
    contrib = col * part

    @pl.when((e == 0) & (i == 0))
    def _init():
        o_ref[...] = h1_ref[...] + contrib

    @pl.when((e > 0) | (i > 0))
    def _acc():
        o_ref[...] += contrib


def kernel(h, ln1_w, ln2_w, wq, wk, wv, wo, gate_w, gate_b, w1, w2, w3):
    f32 = jnp.float32
    inv = 1.0 / (THETA ** (np.arange(0, HD, 2, dtype=np.float32) / HD))
    t = np.arange(S, dtype=np.float32)
    f_a = jnp.asarray(np.outer(t, inv), dtype=f32)
    cos_a, sin_a = jnp.cos(f_a), jnp.sin(f_a)
    cq = jnp.tile(cos_a, (1, NH * 2))
    sq = jnp.tile(sin_a, (1, NH * 2))
    ck = jnp.tile(cos_a, (1, NKV * 2))
    sk = jnp.tile(sin_a, (1, NKV * 2))

    def _perm(nh):
        base = np.arange(nh)[:, None] * HD + np.arange(32)[None, :]
        return np.concatenate([base.ravel(), (base + 32).ravel()])

    wq_p = wq[_perm(NH)]
    wk_p = wk[_perm(NKV)]

    nb1 = S // _BS1
    q, k, v = pl.pallas_call(
        _qkv_body,
        grid=(nb1,),
        in_specs=[
            pl.BlockSpec((_BS1, H), lambda i: (i, 0)),
            pl.BlockSpec((1, H), lambda i: (0, 0)),
            pl.BlockSpec((_BS1, NH * HD), lambda i: (i, 0)),
            pl.BlockSpec((_BS1, NH * HD), lambda i: (i, 0)),
            pl.BlockSpec((_BS1, NKV * HD), lambda i: (i, 0)),
            pl.BlockSpec((_BS1, NKV * HD), lambda i: (i, 0)),
            pl.BlockSpec((NH * HD, H), lambda i: (0, 0)),
            pl.BlockSpec((NKV * HD, H), lambda i: (0, 0)),
            pl.BlockSpec((NKV * HD, H), lambda i: (0, 0)),
        ],
        out_specs=[
            pl.BlockSpec((_BS1, NH * HD), lambda i: (i, 0)),
            pl.BlockSpec((_BS1, NKV * HD), lambda i: (i, 0)),
            pl.BlockSpec((_BS1, NKV * HD), lambda i: (i, 0)),
        ],
        out_shape=[
            jax.ShapeDtypeStruct((S, NH * HD), f32),
            jax.ShapeDtypeStruct((S, NKV * HD), f32),
            jax.ShapeDtypeStruct((S, NKV * HD), f32),
        ],
    )(h, ln1_w.reshape(1, H), cq, sq, ck, sk, wq_p, wk_p, wv)

    qh = q.reshape(S, 2, NH, 32).transpose(2, 0, 1, 3).reshape(NH, S, HD)
    kh = k.reshape(S, 2, NKV, 32).transpose(2, 0, 1, 3).reshape(NKV, S, HD)
    vh = v.reshape(S, NKV, HD).transpose(1, 0, 2)

    rep = NH // NKV
    ctx = pl.pallas_call(
        _attn_body,
        grid=(NH, S // _BQ),
        in_specs=[
            pl.BlockSpec((1, _BQ, HD), lambda hh, qb: (hh, qb, 0)),
            pl.BlockSpec((1, S, HD), lambda hh, qb: (hh // rep, 0, 0)),
            pl.BlockSpec((1, S, HD), lambda hh, qb: (hh // rep, 0, 0)),
        ],
        out_specs=pl.BlockSpec((1, _BQ, HD), lambda hh, qb: (hh, qb, 0)),
        out_shape=jax.ShapeDtypeStruct((NH, S, HD), f32),
    )(qh, kh, vh)

    ctx2 = ctx.transpose(1, 0, 2).reshape(S, NH * HD)

    nb3 = S // _BS3
    h1, x2, pw = pl.pallas_call(
        _out_router_body,
        grid=(nb3,),
        in_specs=[
            pl.BlockSpec((_BS3, NH * HD), lambda i: (i, 0)),
            pl.BlockSpec((_BS3, H), lambda i: (i, 0)),
            pl.BlockSpec((H, NH * HD), lambda i: (0, 0)),
            pl.BlockSpec((1, H), lambda i: (0, 0)),
            pl.BlockSpec((E, H), lambda i: (0, 0)),
            pl.BlockSpec((1, E), lambda i: (0, 0)),
        ],
        out_specs=[
            pl.BlockSpec((_BS3, H), lambda i: (i, 0)),
            pl.BlockSpec((_BS3, H), lambda i: (i, 0)),
            pl.BlockSpec((_BS3, E), lambda i: (i, 0)),
        ],
        out_shape=[
            jax.ShapeDtypeStruct((S, H), f32),
            jax.ShapeDtypeStruct((S, H), f32),
            jax.ShapeDtypeStruct((S, E), f32),
        ],
    )(ctx2, h, wo, ln2_w.reshape(1, H), gate_w, gate_b.reshape(1, E))

    out = pl.pallas_call(
        _moe_body,
        grid=(E, I // _BI),
        in_specs=[
            pl.BlockSpec((S, H), lambda e, i: (0, 0)),
            pl.BlockSpec((S, H), lambda e, i: (0, 0)),
            pl.BlockSpec((S, E), lambda e, i: (0, 0)),
            pl.BlockSpec((1, _BI, H), lambda e, i: (e, i, 0)),
            pl.BlockSpec((1, _BI, H), lambda e, i: (e, i, 0)),
            pl.BlockSpec((1, H, _BI), lambda e, i: (e, 0, i)),
        ],
        out_specs=pl.BlockSpec((S, H), lambda e, i: (0, 0)),
        out_shape=jax.ShapeDtypeStruct((S, H), f32),
    )(x2, h1, pw, w1, w3, w2)
    return out
